# baseline (speedup 1.0000x reference)
import functools

import jax
import jax.numpy as jnp
from jax import lax
from jax.experimental import pallas as pl
from jax.experimental.pallas import tpu as pltpu
from jax.experimental.pallas import tpu_sc as plsc

NC = 2
NS = 16
NW = NC * NS
CHUNK = 100
GC = 4


def kernel(input_ids, embed_table):
    B, S = input_ids.shape
    V, D = embed_table.shape
    N = B * S
    assert N % (NW * CHUNK * GC * 2) == 0
    b_per_w = N // NW
    n_chunks = b_per_w // CHUNK
    n_groups = n_chunks // GC
    n_pairs = n_groups // 2
    grows = GC * CHUNK

    ids3 = input_ids.reshape(NW, n_chunks, CHUNK).astype(jnp.int32)

    mesh = plsc.VectorSubcoreMesh(
        core_axis_name="c", subcore_axis_name="s",
        num_cores=NC, num_subcores=NS)

    @functools.partial(
        pl.kernel,
        out_type=jax.ShapeDtypeStruct((N, D), jnp.float32),
        mesh=mesh,
        scratch_types=[
            pltpu.VMEM((n_chunks, CHUNK), jnp.int32),
            pltpu.VMEM((grows, D), jnp.float32),
            pltpu.VMEM((grows, D), jnp.float32),
            pltpu.SemaphoreType.DMA((GC,)),
            pltpu.SemaphoreType.DMA((GC,)),
            pltpu.SemaphoreType.DMA,
            pltpu.SemaphoreType.DMA,
        ],
    )
    def gather_kernel(ids_hbm, table_hbm, out_hbm, idx_v,
                      buf_a, buf_b, gsem_a, gsem_b, ssem_a, ssem_b):
        wid = lax.axis_index("s") * NC + lax.axis_index("c")
        base = wid * b_per_w
        pltpu.sync_copy(ids_hbm.at[wid], idx_v)

        def fire_group(g, buf, gsem):
            return [
                pltpu.async_copy(
                    table_hbm.at[idx_v.at[g * GC + k]],
                    buf.at[pl.ds(k * CHUNK, CHUNK)], gsem.at[k])
                for k in range(GC)
            ]

        def fire_s(g, buf, ssem):
            pltpu.async_copy(
                buf, out_hbm.at[pl.ds(base + g * grows, grows)], ssem)

        def drain_s(g, buf, ssem):
            pltpu.make_async_copy(
                buf, out_hbm.at[pl.ds(base + g * grows, grows)], ssem).wait()

        hga = fire_group(0, buf_a, gsem_a)
        hgb = fire_group(1, buf_b, gsem_b)
        for h in hga:
            h.wait()
        fire_s(0, buf_a, ssem_a)
        for h in hgb:
            h.wait()
        fire_s(1, buf_b, ssem_b)

        @pl.loop(1, n_pairs)
        def body(m):
            g0 = 2 * m
            g1 = g0 + 1
            drain_s(g0 - 2, buf_a, ssem_a)
            hga = fire_group(g0, buf_a, gsem_a)
            drain_s(g1 - 2, buf_b, ssem_b)
            hgb = fire_group(g1, buf_b, gsem_b)
            for h in hga:
                h.wait()
            fire_s(g0, buf_a, ssem_a)
            for h in hgb:
                h.wait()
            fire_s(g1, buf_b, ssem_b)

        drain_s(n_groups - 2, buf_a, ssem_a)
        drain_s(n_groups - 1, buf_b, ssem_b)

    out = gather_kernel(ids3, embed_table)
    return out.reshape(B, S, D)
